# Initial kernel scaffold; baseline (speedup 1.0000x reference)
import jax
import jax.numpy as jnp
from jax.experimental import pallas as pl


def kernel(X, shuffled_idx, rand_idx):
    raise NotImplementedError("write your pallas kernel here")


# speedup vs baseline: 3.0888x; 3.0888x over previous
import functools

import jax
import jax.numpy as jnp
from jax import lax
from jax.experimental import pallas as pl
from jax.experimental.pallas import tpu as pltpu
from jax.experimental.pallas import tpu_sc as plsc

B, C, E, P = 16, 8, 256, 1024
ROWS = B * C * E
NC, NS = 2, 16
NW = NC * NS
PANELS_PER_W = (B * C) // NW
R = 16
CHUNKS = E // R
LANES = 16


def _body(x_hbm, idx_hbm, o_hbm, idx_v, in_v, out_v):
    w = lax.axis_index("s") * NC + lax.axis_index("c")

    for k in range(PANELS_PER_W):
        panel = w * PANELS_PER_W + k
        ch = lax.rem(panel, C)
        pltpu.sync_copy(idx_hbm.at[ch], idx_v)

        def chunk_body(j, _, panel=panel):
            base = (panel * E + j * R) * P
            pltpu.sync_copy(x_hbm.at[pl.ds(base, R * P)], in_v)

            def pc_body(pc, _):
                iv = idx_v[pl.ds(pc * LANES, LANES)]
                off = pc * LANES
                for r in range(R):
                    out_v[pl.ds(r * P + off, LANES)] = plsc.load_gather(
                        in_v, [iv + r * P])
                return 0

            lax.fori_loop(0, P // LANES, pc_body, 0)
            pltpu.sync_copy(out_v, o_hbm.at[pl.ds(base, R * P)])
            return 0

        lax.fori_loop(0, CHUNKS, chunk_body, 0)


@jax.jit
def _run(x1d, idx):
    mesh = plsc.VectorSubcoreMesh(core_axis_name="c", subcore_axis_name="s")
    return pl.kernel(
        _body,
        out_type=jax.ShapeDtypeStruct((ROWS * P,), jnp.float32),
        mesh=mesh,
        compiler_params=pltpu.CompilerParams(needs_layout_passes=False),
        scratch_types=[
            pltpu.VMEM((P,), jnp.int32),
            pltpu.VMEM((R * P,), jnp.float32),
            pltpu.VMEM((R * P,), jnp.float32),
        ],
    )(x1d, idx)


def kernel(X, shuffled_idx, rand_idx):
    idx = lax.dynamic_index_in_dim(shuffled_idx, rand_idx, 0, keepdims=False)
    out = _run(X.reshape(ROWS * P), idx)
    return out.reshape(B, C, E, P)


# speedup vs baseline: 6.0982x; 1.9743x over previous
import jax
import jax.numpy as jnp
from jax import lax
from jax.experimental import pallas as pl
from jax.experimental.pallas import tpu as pltpu
from jax.experimental.pallas import tpu_sc as plsc

B, C, E, P = 16, 8, 256, 1024
ROWS = B * C * E
NC, NS = 2, 16
NW = NC * NS
PANELS_PER_W = (B * C) // NW
R = 16
NCHUNK = PANELS_PER_W * E // R
CW = R * P
LANES = 16
UNROLL = 2


def _body(x_hbm, idx_hbm, o_hbm, idx_v, in0, in1, out0, out1,
          si0, si1, so0, so1):
    w = lax.axis_index("s") * NC + lax.axis_index("c")
    ins, outs, sin, sout = (in0, in1), (out0, out1), (si0, si1), (so0, so1)

    for k in range(PANELS_PER_W):
        ch = lax.rem(w * PANELS_PER_W + k, C)
        pltpu.sync_copy(idx_hbm.at[ch], idx_v.at[pl.ds(k * P, P)])

    wbase = w * NCHUNK

    def start_in(g, b):
        pltpu.async_copy(x_hbm.at[pl.ds((wbase + g) * CW, CW)], ins[b], sin[b])

    def start_out(g, b):
        pltpu.async_copy(outs[b], o_hbm.at[pl.ds((wbase + g) * CW, CW)],
                         sout[b])

    def wait_in(b):
        pltpu.make_async_copy(x_hbm.at[pl.ds(0, CW)], ins[b], sin[b]).wait()

    def wait_out(b):
        pltpu.make_async_copy(outs[b], o_hbm.at[pl.ds(0, CW)], sout[b]).wait()

    def gather(g, b):
        ibase = lax.div(g, E // R) * P

        @plsc.parallel_loop(0, P // LANES, unroll=UNROLL)
        def _(pc):
            off = pc * LANES
            iv = idx_v[pl.ds(ibase + off, LANES)]
            for r in range(R):
                outs[b][pl.ds(r * P + off, LANES)] = plsc.load_gather(
                    ins[b], [iv + r * P])

    start_in(0, 0)
    start_in(1, 1)

    def loop_body(j, _):
        for b in range(2):
            g = j * 2 + b
            wait_in(b)
            pl.when(j >= 1)(lambda b=b: wait_out(b))
            gather(g, b)
            start_out(g, b)
            pl.when(j < NCHUNK // 2 - 1)(lambda g=g, b=b: start_in(g + 2, b))
        return 0

    lax.fori_loop(0, NCHUNK // 2, loop_body, 0)
    wait_out(0)
    wait_out(1)


@jax.jit
def _run(x1d, idx):
    mesh = plsc.VectorSubcoreMesh(core_axis_name="c", subcore_axis_name="s")
    return pl.kernel(
        _body,
        out_type=jax.ShapeDtypeStruct((ROWS * P,), jnp.float32),
        mesh=mesh,
        compiler_params=pltpu.CompilerParams(needs_layout_passes=False),
        scratch_types=[
            pltpu.VMEM((PANELS_PER_W * P,), jnp.int32),
            pltpu.VMEM((CW,), jnp.float32),
            pltpu.VMEM((CW,), jnp.float32),
            pltpu.VMEM((CW,), jnp.float32),
            pltpu.VMEM((CW,), jnp.float32),
            pltpu.SemaphoreType.DMA,
            pltpu.SemaphoreType.DMA,
            pltpu.SemaphoreType.DMA,
            pltpu.SemaphoreType.DMA,
        ],
    )(x1d, idx)


def kernel(X, shuffled_idx, rand_idx):
    idx = lax.dynamic_index_in_dim(shuffled_idx, rand_idx, 0, keepdims=False)
    out = _run(X.reshape(ROWS * P), idx)
    return out.reshape(B, C, E, P)


# speedup vs baseline: 17.6865x; 2.9003x over previous
import jax
import jax.numpy as jnp
from jax import lax
from jax.experimental import pallas as pl
from jax.experimental.pallas import tpu as pltpu
from jax.experimental.pallas import tpu_sc as plsc

B, C, E, P = 16, 8, 256, 1024
ROWS = B * C * E
NC, NS = 2, 16
NW = NC * NS
PANELS_PER_W = (B * C) // NW
R = 16
NCHUNK = PANELS_PER_W * E // R
CW = R * P
LANES = 16
UNROLL = 2


def _body(x_hbm, idx_hbm, o_hbm, idx_v, in0, in1, out0, out1,
          si0, si1, so0, so1):
    w = lax.axis_index("s") * NC + lax.axis_index("c")
    ins, outs, sin, sout = (in0, in1), (out0, out1), (si0, si1), (so0, so1)

    for k in range(PANELS_PER_W):
        ch = lax.rem(w * PANELS_PER_W + k, C)
        pltpu.sync_copy(idx_hbm.at[pl.ds(ch * P, P)],
                        idx_v.at[pl.ds(k * P, P)])

    wbase = w * NCHUNK

    def start_in(g, b):
        pltpu.async_copy(x_hbm.at[pl.ds((wbase + g) * R, R)], ins[b], sin[b])

    def start_out(g, b):
        pltpu.async_copy(outs[b], o_hbm.at[pl.ds((wbase + g) * R, R)],
                         sout[b])

    def wait_in(b):
        pltpu.make_async_copy(x_hbm.at[pl.ds(0, R)], ins[b], sin[b]).wait()

    def wait_out(b):
        pltpu.make_async_copy(outs[b], o_hbm.at[pl.ds(0, R)], sout[b]).wait()

    def gather(g, b):
        ibase = lax.div(g, E // R) * P

        @plsc.parallel_loop(0, P // LANES, unroll=UNROLL)
        def _(pc):
            off = pc * LANES
            iv = idx_v[pl.ds(ibase + off, LANES)]
            for r in range(R):
                rv = jnp.full((LANES,), r, jnp.int32)
                outs[b][r, pl.ds(off, LANES)] = plsc.load_gather(
                    ins[b], [rv, iv])

    start_in(0, 0)
    start_in(1, 1)

    def loop_body(j, _):
        for b in range(2):
            g = j * 2 + b
            wait_in(b)
            pl.when(j >= 1)(lambda b=b: wait_out(b))
            gather(g, b)
            start_out(g, b)
            pl.when(j < NCHUNK // 2 - 1)(lambda g=g, b=b: start_in(g + 2, b))
        return 0

    lax.fori_loop(0, NCHUNK // 2, loop_body, 0)
    wait_out(0)
    wait_out(1)


@jax.jit
def _run(x2d, idx_flat):
    mesh = plsc.VectorSubcoreMesh(core_axis_name="c", subcore_axis_name="s")
    return pl.kernel(
        _body,
        out_type=jax.ShapeDtypeStruct((ROWS, P), jnp.float32),
        mesh=mesh,
        compiler_params=pltpu.CompilerParams(needs_layout_passes=False,
                                             use_tc_tiling_on_sc=True),
        scratch_types=[
            pltpu.VMEM((PANELS_PER_W * P,), jnp.int32),
            pltpu.VMEM((R, P), jnp.float32),
            pltpu.VMEM((R, P), jnp.float32),
            pltpu.VMEM((R, P), jnp.float32),
            pltpu.VMEM((R, P), jnp.float32),
            pltpu.SemaphoreType.DMA,
            pltpu.SemaphoreType.DMA,
            pltpu.SemaphoreType.DMA,
            pltpu.SemaphoreType.DMA,
        ],
    )(x2d, idx_flat)


def kernel(X, shuffled_idx, rand_idx):
    idx = lax.dynamic_index_in_dim(shuffled_idx, rand_idx, 0, keepdims=False)
    out = _run(X.reshape(ROWS, P), idx.reshape(C * P))
    return out.reshape(B, C, E, P)


# speedup vs baseline: 18.1079x; 1.0238x over previous
import jax
import jax.numpy as jnp
from jax import lax
from jax.experimental import pallas as pl
from jax.experimental.pallas import tpu as pltpu
from jax.experimental.pallas import tpu_sc as plsc

B, C, E, P = 16, 8, 256, 1024
ROWS = B * C * E
NC, NS = 2, 16
NW = NC * NS
PANELS_PER_W = (B * C) // NW
R = 16
NCHUNK = PANELS_PER_W * E // R
NBUF = 3
LANES = 16
UNROLL = 4


def _body(x_hbm, idx_hbm, o_hbm, idx_v, in0, in1, in2, out0, out1, out2,
          si0, si1, si2, so0, so1, so2):
    w = lax.axis_index("s") * NC + lax.axis_index("c")
    ins, outs = (in0, in1, in2), (out0, out1, out2)
    sin, sout = (si0, si1, si2), (so0, so1, so2)

    for k in range(PANELS_PER_W):
        ch = lax.rem(w * PANELS_PER_W + k, C)
        pltpu.sync_copy(idx_hbm.at[pl.ds(ch * P, P)],
                        idx_v.at[pl.ds(k * P, P)])

    wbase = w * NCHUNK

    def start_in(g, b):
        pltpu.async_copy(x_hbm.at[pl.ds((wbase + g) * R, R)], ins[b], sin[b])

    def start_out(g, b):
        pltpu.async_copy(outs[b], o_hbm.at[pl.ds((wbase + g) * R, R)],
                         sout[b])

    def wait_in(b):
        pltpu.make_async_copy(x_hbm.at[pl.ds(0, R)], ins[b], sin[b]).wait()

    def wait_out(b):
        pltpu.make_async_copy(outs[b], o_hbm.at[pl.ds(0, R)], sout[b]).wait()

    def gather(g, b):
        ibase = lax.div(g, E // R) * P

        @plsc.parallel_loop(0, P // LANES, unroll=UNROLL)
        def _(pc):
            off = pc * LANES
            iv = idx_v[pl.ds(ibase + off, LANES)]
            for r in range(R):
                rv = jnp.full((LANES,), r, jnp.int32)
                outs[b][r, pl.ds(off, LANES)] = plsc.load_gather(
                    ins[b], [rv, iv])

    for b in range(NBUF):
        start_in(b, b)
    wait_in(0)
    gather(0, 0)
    start_out(0, 0)
    start_in(NBUF, 0)

    def loop_body(j, _):
        for i in range(NBUF):
            g = 1 + j * NBUF + i
            b = (1 + i) % NBUF
            wait_in(b)
            if i == NBUF - 1:
                wait_out(b)
            else:
                pl.when(j >= 1)(lambda b=b: wait_out(b))
            gather(g, b)
            start_out(g, b)
            pl.when(j < (NCHUNK - 1) // NBUF - 1)(
                lambda g=g, b=b: start_in(g + NBUF, b))
        return 0

    lax.fori_loop(0, (NCHUNK - 1) // NBUF, loop_body, 0)
    for b in (1, 2, 0):
        wait_out(b)


@jax.jit
def _run(x2d, idx_flat):
    mesh = plsc.VectorSubcoreMesh(core_axis_name="c", subcore_axis_name="s")
    return pl.kernel(
        _body,
        out_type=jax.ShapeDtypeStruct((ROWS, P), jnp.float32),
        mesh=mesh,
        compiler_params=pltpu.CompilerParams(needs_layout_passes=False,
                                             use_tc_tiling_on_sc=True),
        scratch_types=[
            pltpu.VMEM((PANELS_PER_W * P,), jnp.int32),
            pltpu.VMEM((R, P), jnp.float32),
            pltpu.VMEM((R, P), jnp.float32),
            pltpu.VMEM((R, P), jnp.float32),
            pltpu.VMEM((R, P), jnp.float32),
            pltpu.VMEM((R, P), jnp.float32),
            pltpu.VMEM((R, P), jnp.float32),
            pltpu.SemaphoreType.DMA,
            pltpu.SemaphoreType.DMA,
            pltpu.SemaphoreType.DMA,
            pltpu.SemaphoreType.DMA,
            pltpu.SemaphoreType.DMA,
            pltpu.SemaphoreType.DMA,
        ],
    )(x2d, idx_flat)


def kernel(X, shuffled_idx, rand_idx):
    idx = lax.dynamic_index_in_dim(shuffled_idx, rand_idx, 0, keepdims=False)
    out = _run(X.reshape(ROWS, P), idx.reshape(C * P))
    return out.reshape(B, C, E, P)
